# baseline (speedup 1.0000x reference)
import jax
import jax.numpy as jnp
from jax.experimental import pallas as pl
from jax.experimental.pallas import tpu as pltpu

_BM = 16384


def _mlp_knots_kernel(x_ref, w1t_ref, b1_ref, w2t_ref, b2_ref,
                      w3t_ref, b3_ref, out_ref):
    f32 = jnp.float32
    x = x_ref[0]

    h1 = jnp.dot(x, w1t_ref[...], preferred_element_type=f32)
    h1 = jnp.maximum(h1 + b1_ref[...], 0.0)

    h2 = jnp.dot(h1, w2t_ref[...], preferred_element_type=f32)
    h2 = jnp.maximum(h2 + b2_ref[...], 0.0)

    lg = jnp.dot(h2, w3t_ref[...], preferred_element_type=f32)
    lg = lg + b3_ref[...]

    m = jnp.max(lg, axis=1, keepdims=True)
    e = jnp.exp(lg - m)

    r5 = jax.lax.broadcasted_iota(jnp.int32, (5, 16), 0)
    c16 = jax.lax.broadcasted_iota(jnp.int32, (5, 16), 1)
    ct = (((c16 >= 4) & (c16 < 8) & (r5 <= (c16 - 4)))
          | ((c16 >= 8) & (c16 < 12))).astype(f32)

    u = jnp.dot(e, ct, preferred_element_type=f32)
    u = u * (1.0 / u[:, 8:9])
    out_ref[...] = u[:, :12]


@jax.jit
def kernel(input, W1, b1, W2, b2, W3, b3):
    B = input.shape[1]
    f32 = jnp.float32
    w1t = W1.T
    w2t = W2.T
    w3t = W3.T
    b1r = b1.reshape(1, -1)
    b2r = b2.reshape(1, -1)
    b3r = b3.reshape(1, -1)

    out = pl.pallas_call(
        _mlp_knots_kernel,
        grid=(B // _BM,),
        in_specs=[
            pl.BlockSpec((1, _BM, 8), lambda i: (0, i, 0)),
            pl.BlockSpec((8, 128), lambda i: (0, 0)),
            pl.BlockSpec((1, 128), lambda i: (0, 0)),
            pl.BlockSpec((128, 128), lambda i: (0, 0)),
            pl.BlockSpec((1, 128), lambda i: (0, 0)),
            pl.BlockSpec((128, 5), lambda i: (0, 0)),
            pl.BlockSpec((1, 5), lambda i: (0, 0)),
        ],
        out_specs=pl.BlockSpec((_BM, 12), lambda i: (i, 0)),
        out_shape=jax.ShapeDtypeStruct((B, 12), f32),
        compiler_params=pltpu.CompilerParams(
            dimension_semantics=("arbitrary",),
        ),
    )(input, w1t, b1r, w2t, b2r, w3t, b3r)
    return out


# speedup vs baseline: 1.5733x; 1.5733x over previous
import jax
import jax.numpy as jnp
from jax.experimental import pallas as pl
from jax.experimental.pallas import tpu as pltpu

_BM = 2048


def _mlp_knots_kernel(x_ref, w1t_ref, b1_ref, w2t_ref, b2_ref,
                      w3t_ref, b3_ref, out_ref):
    f32 = jnp.float32
    x = x_ref[0]

    h1 = jnp.dot(x, w1t_ref[...], preferred_element_type=f32)
    h1 = jnp.maximum(h1 + b1_ref[...], 0.0)

    h2 = jnp.dot(h1, w2t_ref[...], preferred_element_type=f32)
    h2 = jnp.maximum(h2 + b2_ref[...], 0.0)

    lg = jnp.dot(h2, w3t_ref[...], preferred_element_type=f32)
    lg = lg + b3_ref[...]

    m = jnp.max(lg, axis=1, keepdims=True)
    e = jnp.exp(lg - m)

    r5 = jax.lax.broadcasted_iota(jnp.int32, (5, 16), 0)
    c16 = jax.lax.broadcasted_iota(jnp.int32, (5, 16), 1)
    ct = (((c16 >= 4) & (c16 < 8) & (r5 <= (c16 - 4)))
          | ((c16 >= 8) & (c16 < 12))).astype(f32)

    u = jnp.dot(e, ct, preferred_element_type=f32)
    u = u * (1.0 / u[:, 8:9])
    out_ref[...] = u[:, :12]


@jax.jit
def kernel(input, W1, b1, W2, b2, W3, b3):
    B = input.shape[1]
    f32 = jnp.float32
    w1t = W1.T
    w2t = W2.T
    w3t = W3.T
    b1r = b1.reshape(1, -1)
    b2r = b2.reshape(1, -1)
    b3r = b3.reshape(1, -1)

    out = pl.pallas_call(
        _mlp_knots_kernel,
        grid=(B // _BM,),
        in_specs=[
            pl.BlockSpec((1, _BM, 8), lambda i: (0, i, 0)),
            pl.BlockSpec((8, 128), lambda i: (0, 0)),
            pl.BlockSpec((1, 128), lambda i: (0, 0)),
            pl.BlockSpec((128, 128), lambda i: (0, 0)),
            pl.BlockSpec((1, 128), lambda i: (0, 0)),
            pl.BlockSpec((128, 5), lambda i: (0, 0)),
            pl.BlockSpec((1, 5), lambda i: (0, 0)),
        ],
        out_specs=pl.BlockSpec((_BM, 12), lambda i: (i, 0)),
        out_shape=jax.ShapeDtypeStruct((B, 12), f32),
        compiler_params=pltpu.CompilerParams(
            dimension_semantics=("parallel",),
        ),
    )(input, w1t, b1r, w2t, b2r, w3t, b3r)
    return out


# speedup vs baseline: 1.6833x; 1.0699x over previous
import jax
import jax.numpy as jnp
from jax.experimental import pallas as pl
from jax.experimental.pallas import tpu as pltpu

_BM = 2048

_NT = (((1,), (1,)), ((), ()))
_NN = (((1,), (0,)), ((), ()))


def _mlp_knots_kernel(x_ref, w1_ref, b1_ref, w2_ref, b2_ref,
                      w3_ref, b3_ref, out_ref):
    f32 = jnp.float32
    x = x_ref[0]

    h1 = jax.lax.dot_general(w1_ref[...], x, _NT,
                             preferred_element_type=f32)
    h1 = jnp.maximum(h1 + b1_ref[...], 0.0)

    h2 = jax.lax.dot_general(w2_ref[...], h1, _NN,
                             preferred_element_type=f32)
    h2 = jnp.maximum(h2 + b2_ref[...], 0.0)

    lg = jax.lax.dot_general(w3_ref[...], h2, _NN,
                             preferred_element_type=f32)
    lg = lg + b3_ref[...]
    m = jnp.max(lg, axis=0, keepdims=True)
    e = jnp.exp(lg - m)

    r16 = jax.lax.broadcasted_iota(jnp.int32, (16, 5), 0)
    k5 = jax.lax.broadcasted_iota(jnp.int32, (16, 5), 1)
    ct = (((r16 >= 4) & (r16 < 8) & (k5 <= (r16 - 4)))
          | ((r16 >= 8) & (r16 < 12))).astype(f32)

    u = jax.lax.dot_general(ct, e, _NN,
                            preferred_element_type=f32)
    u = u * (1.0 / u[8:9, :])
    ut = jnp.transpose(u, (1, 0))
    out_ref[...] = ut[:, :12]


@jax.jit
def kernel(input, W1, b1, W2, b2, W3, b3):
    B = input.shape[1]
    f32 = jnp.float32
    b1c = b1.reshape(-1, 1)
    b2c = b2.reshape(-1, 1)
    b3c = b3.reshape(-1, 1)

    out = pl.pallas_call(
        _mlp_knots_kernel,
        grid=(B // _BM,),
        in_specs=[
            pl.BlockSpec((1, _BM, 8), lambda i: (0, i, 0)),
            pl.BlockSpec((128, 8), lambda i: (0, 0)),
            pl.BlockSpec((128, 1), lambda i: (0, 0)),
            pl.BlockSpec((128, 128), lambda i: (0, 0)),
            pl.BlockSpec((128, 1), lambda i: (0, 0)),
            pl.BlockSpec((5, 128), lambda i: (0, 0)),
            pl.BlockSpec((5, 1), lambda i: (0, 0)),
        ],
        out_specs=pl.BlockSpec((_BM, 12), lambda i: (i, 0)),
        out_shape=jax.ShapeDtypeStruct((B, 12), f32),
        compiler_params=pltpu.CompilerParams(
            dimension_semantics=("parallel",),
        ),
    )(input, W1, b1c, W2, b2c, W3, b3c)
    return out
